# baseline (speedup 1.0000x reference)
import functools

import jax
import jax.numpy as jnp
from jax.experimental import pallas as pl

_B = 8
_A = 8400
_NC = 80
_NGT = 32
_IMG = 640.0
_LB = 7.5
_LC = 0.5
_LAUX = 0.25
_NCAND = 50
_EPS = 1e-07
_INF_BITS = 0x7F800000


def _atan(x):
    ax = jnp.abs(x)
    t3 = ax > 2.414213562373095
    t2 = ax > 0.4142135623730951
    x1 = jnp.where(t3, -1.0 / ax, jnp.where(t2, (ax - 1.0) / (ax + 1.0), ax))
    y0 = jnp.where(t3, jnp.pi / 2, jnp.where(t2, jnp.pi / 4, 0.0))
    z = x1 * x1
    p = ((((8.05374449538e-2 * z - 1.38776856032e-1) * z
           + 1.99777106478e-1) * z - 3.33329491539e-1) * z * x1 + x1)
    r = y0 + p
    return jnp.where(x < 0.0, -r, r)


def _softplus(x):
    return jnp.maximum(x, 0.0) + jnp.log1p(jnp.exp(-jnp.abs(x)))


def _kth_smallest_bits(bits, k):
    g = bits.shape[0]
    lo = jnp.zeros((g, 1), jnp.int32)
    hi = jnp.full((g, 1), _INF_BITS, jnp.int32)

    def body(_, carry):
        lo, hi = carry
        mid = lo + (hi - lo) // 2
        cnt = jnp.sum((bits <= mid).astype(jnp.int32), axis=1, keepdims=True)
        take = cnt >= k
        return jnp.where(take, lo, mid + 1), jnp.where(take, mid, hi)

    lo, hi = jax.lax.fori_loop(0, 31, body, (lo, hi))
    return hi


def _loss_step(box_ref, log_ref, gt_ref, lbl_ref, out_ref, *, k_match):
    cx = box_ref[0, 0:1, :]
    cy = box_ref[0, 1:2, :]
    w = box_ref[0, 2:3, :]
    h = box_ref[0, 3:4, :]
    px1 = cx - 0.5 * w
    py1 = cy - 0.5 * h
    px2 = cx + 0.5 * w
    py2 = cy + 0.5 * h

    logits = log_ref[0][:, 4:]

    gt = gt_ref[0]
    scale = jnp.where(jnp.max(gt) <= 1.01, _IMG, 1.0)
    gx1 = gt[:, 0:1] * scale
    gy1 = gt[:, 1:2] * scale
    gx2 = gt[:, 2:3] * scale
    gy2 = gt[:, 3:4] * scale

    gcx = 0.5 * (gx1 + gx2)
    gcy = 0.5 * (gy1 + gy2)
    pcx = 0.5 * (px1 + px2)
    pcy = 0.5 * (py1 + py2)
    dist = (pcx - gcx) ** 2 + (pcy - gcy) ** 2

    dist_bits = jax.lax.bitcast_convert_type(dist, jnp.int32)
    t50 = _kth_smallest_bits(dist_bits, _NCAND)
    cand = jnp.any(dist_bits <= t50, axis=0, keepdims=True)

    lbl = lbl_ref[0]
    onehot_l = (jax.lax.broadcasted_iota(jnp.int32, (_NGT, _NC), 1) == lbl)
    logit_sel = jax.lax.dot_general(
        onehot_l.astype(jnp.float32), logits,
        dimension_numbers=(((1,), (1,)), ((), ())),
        preferred_element_type=jnp.float32)
    cost_cls = _softplus(-logit_sel)

    area_p = jnp.maximum(px2 - px1, 0.0) * jnp.maximum(py2 - py1, 0.0)
    area_g = jnp.maximum(gx2 - gx1, 0.0) * jnp.maximum(gy2 - gy1, 0.0)
    iw = jnp.maximum(jnp.minimum(px2, gx2) - jnp.maximum(px1, gx1), 0.0)
    ih = jnp.maximum(jnp.minimum(py2, gy2) - jnp.maximum(py1, gy1), 0.0)
    inter = iw * ih
    union = area_p + area_g - inter + _EPS
    iou = inter / union

    total_cost = cost_cls + 6.0 * (1.0 - iou) + 1e-06
    masked = jnp.where(cand, total_cost, jnp.inf)

    if k_match == 1:
        rowmin = jnp.min(masked, axis=1, keepdims=True)
        matches = masked <= rowmin
    else:
        cost_bits = jax.lax.bitcast_convert_type(masked, jnp.int32)
        tk = _kth_smallest_bits(cost_bits, k_match)
        matches = cost_bits <= tk

    colmin = jnp.min(masked, axis=0, keepdims=True)
    g_iota = jax.lax.broadcasted_iota(jnp.int32, (_NGT, _A), 0)
    gt_ind = jnp.min(jnp.where(masked == colmin, g_iota, _NGT),
                     axis=0, keepdims=True)

    anym = jnp.any(matches, axis=0, keepdims=True)
    if k_match == 1:
        msum = jnp.sum(matches.astype(jnp.float32), axis=0, keepdims=True)
        dup = jnp.max(msum) > 1.0
        validf = jnp.where(dup, cand.astype(jnp.float32),
                           anym.astype(jnp.float32))
    else:
        validf = anym.astype(jnp.float32)

    onehot_g = (g_iota == gt_ind).astype(jnp.float32)
    tx1 = jnp.sum(onehot_g * gx1, axis=0, keepdims=True)
    ty1 = jnp.sum(onehot_g * gy1, axis=0, keepdims=True)
    tx2 = jnp.sum(onehot_g * gx2, axis=0, keepdims=True)
    ty2 = jnp.sum(onehot_g * gy2, axis=0, keepdims=True)
    sel_logit = jnp.sum(onehot_g * logit_sel, axis=0, keepdims=True)

    iw2 = jnp.maximum(jnp.minimum(px2, tx2) - jnp.maximum(px1, tx1), 0.0)
    ih2 = jnp.maximum(jnp.minimum(py2, ty2) - jnp.maximum(py1, ty1), 0.0)
    inter2 = iw2 * ih2
    w1 = px2 - px1
    h1 = py2 - py1
    w2 = tx2 - tx1
    h2 = ty2 - ty1
    union2 = w1 * h1 + w2 * h2 - inter2 + _EPS
    iou2 = inter2 / union2
    cw = jnp.maximum(px2, tx2) - jnp.minimum(px1, tx1)
    ch = jnp.maximum(py2, ty2) - jnp.minimum(py1, ty1)
    c2 = cw ** 2 + ch ** 2 + _EPS
    rho2 = ((px1 + px2 - tx1 - tx2) ** 2 + (py1 + py2 - ty1 - ty2) ** 2) / 4.0
    v = 4.0 / 3.14159 ** 2 * (_atan(w2 / (h2 + _EPS))
                              - _atan(w1 / (h1 + _EPS))) ** 2
    alpha = v / (1.0 - iou2 + v + _EPS)
    ciou = jnp.clip(iou2 - rho2 / c2 - alpha * v, -1.0, 1.0)

    loss_box = jnp.sum(validf * (1.0 - ciou))
    total_pos = jnp.sum(validf)
    loss_cls = jnp.sum(_softplus(logits)) - jnp.sum(validf * sel_logit)

    lane = jax.lax.broadcasted_iota(jnp.int32, (1, 128), 1)
    vec = (jnp.where(lane == 0, loss_box, 0.0)
           + jnp.where(lane == 1, loss_cls, 0.0)
           + jnp.where(lane == 2, total_pos, 0.0))
    out_ref[0] = vec


def _branch_partials(p, gt, lbl, k_match):
    box_t = jnp.transpose(p[..., :4], (0, 2, 1))
    part = pl.pallas_call(
        functools.partial(_loss_step, k_match=k_match),
        grid=(_B,),
        in_specs=[
            pl.BlockSpec((1, 4, _A), lambda i: (i, 0, 0)),
            pl.BlockSpec((1, _A, 4 + _NC), lambda i: (i, 0, 0)),
            pl.BlockSpec((1, _NGT, 4), lambda i: (i, 0, 0)),
            pl.BlockSpec((1, _NGT, 1), lambda i: (i, 0, 0)),
        ],
        out_specs=pl.BlockSpec((1, 1, 128), lambda i: (i, 0, 0)),
        out_shape=jax.ShapeDtypeStruct((_B, 1, 128), jnp.float32),
    )(box_t, p, gt, lbl)
    return part[:, 0, :3]


def kernel(p_main, p_aux, tgt_boxes, tgt_labels):
    lbl = tgt_labels.astype(jnp.int32).reshape(_B, _NGT, 1)
    pm = _branch_partials(p_main, tgt_boxes, lbl, 1)
    pa = _branch_partials(p_aux, tgt_boxes, lbl, 10)
    norm_m = jnp.maximum(1.0, jnp.sum(pm[:, 2]))
    norm_a = jnp.maximum(1.0, jnp.sum(pa[:, 2]))
    lbm = jnp.sum(pm[:, 0]) * _LB / norm_m
    lcm = jnp.sum(pm[:, 1]) * _LC / norm_m
    lba = jnp.sum(pa[:, 0]) * _LB / norm_a
    lca = jnp.sum(pa[:, 1]) * _LC / norm_a
    loss = lbm + lcm + _LAUX * (lba + lca)
    return (loss, lbm, lcm, lba, lca)


# speedup vs baseline: 1.2866x; 1.2866x over previous
import functools

import jax
import jax.numpy as jnp
from jax.experimental import pallas as pl

_B = 8
_A = 8400
_NC = 80
_NGT = 32
_IMG = 640.0
_LB = 7.5
_LC = 0.5
_LAUX = 0.25
_NCAND = 50
_EPS = 1e-07
_INF_BITS = 0x7F800000


def _atan(x):
    ax = jnp.abs(x)
    t3 = ax > 2.414213562373095
    t2 = ax > 0.4142135623730951
    x1 = jnp.where(t3, -1.0 / ax, jnp.where(t2, (ax - 1.0) / (ax + 1.0), ax))
    y0 = jnp.where(t3, jnp.pi / 2, jnp.where(t2, jnp.pi / 4, 0.0))
    z = x1 * x1
    p = ((((8.05374449538e-2 * z - 1.38776856032e-1) * z
           + 1.99777106478e-1) * z - 3.33329491539e-1) * z * x1 + x1)
    r = y0 + p
    return jnp.where(x < 0.0, -r, r)


def _softplus(x):
    return jnp.maximum(x, 0.0) + jnp.log1p(jnp.exp(-jnp.abs(x)))


def _kth_smallest_bits(bits, k):
    g = bits.shape[0]
    lo = jnp.zeros((g, 1), jnp.int32)
    hi = jnp.full((g, 1), _INF_BITS, jnp.int32)

    def body(_, carry):
        lo, hi = carry
        mid = lo + (hi - lo) // 2
        cnt = jnp.sum((bits <= mid).astype(jnp.int32), axis=1, keepdims=True)
        take = cnt >= k
        return jnp.where(take, lo, mid + 1), jnp.where(take, mid, hi)

    lo, hi = jax.lax.fori_loop(0, 31, body, (lo, hi))
    return hi


def _loss_step(p_ref, gt_ref, lbl_ref, out_ref, *, k_match):
    bt = jnp.transpose(p_ref[0][:, 0:8])
    cx = bt[0:1, :]
    cy = bt[1:2, :]
    w = bt[2:3, :]
    h = bt[3:4, :]
    px1 = cx - 0.5 * w
    py1 = cy - 0.5 * h
    px2 = cx + 0.5 * w
    py2 = cy + 0.5 * h

    logits = p_ref[0][:, 4:]

    gt = gt_ref[0]
    scale = jnp.where(jnp.max(gt) <= 1.01, _IMG, 1.0)
    gx1 = gt[:, 0:1] * scale
    gy1 = gt[:, 1:2] * scale
    gx2 = gt[:, 2:3] * scale
    gy2 = gt[:, 3:4] * scale

    gcx = 0.5 * (gx1 + gx2)
    gcy = 0.5 * (gy1 + gy2)
    pcx = 0.5 * (px1 + px2)
    pcy = 0.5 * (py1 + py2)
    dist = (pcx - gcx) ** 2 + (pcy - gcy) ** 2

    dist_bits = jax.lax.bitcast_convert_type(dist, jnp.int32)
    t50 = _kth_smallest_bits(dist_bits, _NCAND)
    cand = jnp.any(dist_bits <= t50, axis=0, keepdims=True)

    lbl = lbl_ref[0]
    onehot_l = (jax.lax.broadcasted_iota(jnp.int32, (_NGT, _NC), 1) == lbl)
    logit_sel = jax.lax.dot_general(
        onehot_l.astype(jnp.float32), logits,
        dimension_numbers=(((1,), (1,)), ((), ())),
        preferred_element_type=jnp.float32)
    cost_cls = _softplus(-logit_sel)

    area_p = jnp.maximum(px2 - px1, 0.0) * jnp.maximum(py2 - py1, 0.0)
    area_g = jnp.maximum(gx2 - gx1, 0.0) * jnp.maximum(gy2 - gy1, 0.0)
    iw = jnp.maximum(jnp.minimum(px2, gx2) - jnp.maximum(px1, gx1), 0.0)
    ih = jnp.maximum(jnp.minimum(py2, gy2) - jnp.maximum(py1, gy1), 0.0)
    inter = iw * ih
    union = area_p + area_g - inter + _EPS
    iou = inter / union

    total_cost = cost_cls + 6.0 * (1.0 - iou) + 1e-06
    masked = jnp.where(cand, total_cost, jnp.inf)

    if k_match == 1:
        rowmin = jnp.min(masked, axis=1, keepdims=True)
        matches = masked <= rowmin
    else:
        def extract(_, carry):
            work, _tk = carry
            m = jnp.min(work, axis=1, keepdims=True)
            return jnp.where(work <= m, jnp.inf, work), m

        _, tk = jax.lax.fori_loop(
            0, k_match, extract, (masked, jnp.zeros((_NGT, 1), jnp.float32)))
        matches = masked <= tk

    colmin = jnp.min(masked, axis=0, keepdims=True)
    g_iota = jax.lax.broadcasted_iota(jnp.int32, (_NGT, _A), 0)
    gt_ind = jnp.min(jnp.where(masked == colmin, g_iota, _NGT),
                     axis=0, keepdims=True)

    anym = jnp.any(matches, axis=0, keepdims=True)
    if k_match == 1:
        msum = jnp.sum(matches.astype(jnp.float32), axis=0, keepdims=True)
        dup = jnp.max(msum) > 1.0
        validf = jnp.where(dup, cand.astype(jnp.float32),
                           anym.astype(jnp.float32))
    else:
        validf = anym.astype(jnp.float32)

    onehot_g = (g_iota == gt_ind).astype(jnp.float32)
    tx1 = jnp.sum(onehot_g * gx1, axis=0, keepdims=True)
    ty1 = jnp.sum(onehot_g * gy1, axis=0, keepdims=True)
    tx2 = jnp.sum(onehot_g * gx2, axis=0, keepdims=True)
    ty2 = jnp.sum(onehot_g * gy2, axis=0, keepdims=True)
    sel_logit = jnp.sum(onehot_g * logit_sel, axis=0, keepdims=True)

    iw2 = jnp.maximum(jnp.minimum(px2, tx2) - jnp.maximum(px1, tx1), 0.0)
    ih2 = jnp.maximum(jnp.minimum(py2, ty2) - jnp.maximum(py1, ty1), 0.0)
    inter2 = iw2 * ih2
    w1 = px2 - px1
    h1 = py2 - py1
    w2 = tx2 - tx1
    h2 = ty2 - ty1
    union2 = w1 * h1 + w2 * h2 - inter2 + _EPS
    iou2 = inter2 / union2
    cw = jnp.maximum(px2, tx2) - jnp.minimum(px1, tx1)
    ch = jnp.maximum(py2, ty2) - jnp.minimum(py1, ty1)
    c2 = cw ** 2 + ch ** 2 + _EPS
    rho2 = ((px1 + px2 - tx1 - tx2) ** 2 + (py1 + py2 - ty1 - ty2) ** 2) / 4.0
    v = 4.0 / 3.14159 ** 2 * (_atan(w2 / (h2 + _EPS))
                              - _atan(w1 / (h1 + _EPS))) ** 2
    alpha = v / (1.0 - iou2 + v + _EPS)
    ciou = jnp.clip(iou2 - rho2 / c2 - alpha * v, -1.0, 1.0)

    loss_box = jnp.sum(validf * (1.0 - ciou))
    total_pos = jnp.sum(validf)
    loss_cls = jnp.sum(_softplus(logits)) - jnp.sum(validf * sel_logit)

    lane = jax.lax.broadcasted_iota(jnp.int32, (1, 128), 1)
    vec = (jnp.where(lane == 0, loss_box, 0.0)
           + jnp.where(lane == 1, loss_cls, 0.0)
           + jnp.where(lane == 2, total_pos, 0.0))
    out_ref[0] = vec


def _branch_partials(p, gt, lbl, k_match):
    part = pl.pallas_call(
        functools.partial(_loss_step, k_match=k_match),
        grid=(_B,),
        in_specs=[
            pl.BlockSpec((1, _A, 4 + _NC), lambda i: (i, 0, 0)),
            pl.BlockSpec((1, _NGT, 4), lambda i: (i, 0, 0)),
            pl.BlockSpec((1, _NGT, 1), lambda i: (i, 0, 0)),
        ],
        out_specs=pl.BlockSpec((1, 1, 128), lambda i: (i, 0, 0)),
        out_shape=jax.ShapeDtypeStruct((_B, 1, 128), jnp.float32),
    )(p, gt, lbl)
    return part[:, 0, :3]


def kernel(p_main, p_aux, tgt_boxes, tgt_labels):
    lbl = tgt_labels.astype(jnp.int32).reshape(_B, _NGT, 1)
    pm = _branch_partials(p_main, tgt_boxes, lbl, 1)
    pa = _branch_partials(p_aux, tgt_boxes, lbl, 10)
    norm_m = jnp.maximum(1.0, jnp.sum(pm[:, 2]))
    norm_a = jnp.maximum(1.0, jnp.sum(pa[:, 2]))
    lbm = jnp.sum(pm[:, 0]) * _LB / norm_m
    lcm = jnp.sum(pm[:, 1]) * _LC / norm_m
    lba = jnp.sum(pa[:, 0]) * _LB / norm_a
    lca = jnp.sum(pa[:, 1]) * _LC / norm_a
    loss = lbm + lcm + _LAUX * (lba + lca)
    return (loss, lbm, lcm, lba, lca)
